# baseline (speedup 1.0000x reference)
import functools

import jax
import jax.numpy as jnp
from jax import lax
from jax.experimental import pallas as pl
from jax.experimental.pallas import tpu as pltpu
from jax.experimental.pallas import tpu_sc as plsc

NC = 2
NS = 16
L = 16
NW = NC * NS
B = 16384
D = 32
BPW = B // NW
NG = BPW // L


def _sc_diff(user_t, item_t, user_ids, pos_ids, neg_ids):
    mesh = plsc.VectorSubcoreMesh(core_axis_name="c", subcore_axis_name="s")

    @functools.partial(
        pl.kernel,
        mesh=mesh,
        compiler_params=pltpu.CompilerParams(
            needs_layout_passes=False, use_tc_tiling_on_sc=True
        ),
        out_type=jax.ShapeDtypeStruct((B,), jnp.float32),
        scratch_types=[
            pltpu.VMEM((BPW,), jnp.int32),
            pltpu.VMEM((BPW,), jnp.int32),
            pltpu.VMEM((BPW,), jnp.int32),
            pltpu.VMEM((4, 4, 8, 128), jnp.float32),
            pltpu.VMEM((4, 4, 8, 128), jnp.float32),
            pltpu.VMEM((4, 4, 8, 128), jnp.float32),
            pltpu.VMEM((BPW,), jnp.float32),
            pltpu.SemaphoreType.DMA((4,)),
            pltpu.SemaphoreType.DMA((4,)),
            pltpu.SemaphoreType.DMA((4,)),
            pltpu.SemaphoreType.DMA,
        ],
    )
    def k(ut, it, uids, pids, nids, out, iu, ip, inn, gu, gp, gn, dv,
          usem, psem, nsem, isem):
        wid = lax.axis_index("s") * NC + lax.axis_index("c")
        base = wid * BPW
        sl = pl.ds(base, BPW)
        pltpu.async_copy(uids.at[sl], iu, isem)
        pltpu.async_copy(pids.at[sl], ip, isem)
        pltpu.async_copy(nids.at[sl], inn, isem)
        pltpu.make_async_copy(uids.at[sl], iu, isem).wait()
        pltpu.make_async_copy(pids.at[sl], ip, isem).wait()
        pltpu.make_async_copy(nids.at[sl], inn, isem).wait()

        def fire(g, b):
            cu = iu[pl.ds(g * L, L)]
            cp = ip[pl.ds(g * L, L)]
            cn = inn[pl.ds(g * L, L)]
            for i in range(L):
                dst = pl.ds(i * 8, 8)
                r0 = (cu[i] // 8) * 8
                pltpu.async_copy(
                    ut.at[:, :, pl.ds(r0, 8)],
                    gu.at[b, :, :, dst], usem.at[b])
                r0 = (cp[i] // 8) * 8
                pltpu.async_copy(
                    it.at[:, :, pl.ds(r0, 8)],
                    gp.at[b, :, :, dst], psem.at[b])
                r0 = (cn[i] // 8) * 8
                pltpu.async_copy(
                    it.at[:, :, pl.ds(r0, 8)],
                    gn.at[b, :, :, dst], nsem.at[b])

        def prefire(g, carry):
            fire(g, lax.rem(g, 4))
            return carry

        lax.fori_loop(0, 3, prefire, 0)
        iota = lax.iota(jnp.int32, L)

        def body(g, carry):
            b = lax.rem(g, 4)

            @pl.when(g < NG - 3)
            def _():
                fire(g + 3, lax.rem(g + 3, 4))

            dummy = ut.at[:, :, pl.ds(0, 128)]
            pltpu.make_async_copy(dummy, gu.at[b], usem.at[b]).wait()
            pltpu.make_async_copy(dummy, gp.at[b], psem.at[b]).wait()
            pltpu.make_async_copy(dummy, gn.at[b], nsem.at[b]).wait()

            bv = jnp.full((L,), b, jnp.int32)
            pos_u = iota * 8 + (iu[pl.ds(g * L, L)] & 7)
            pos_p = iota * 8 + (ip[pl.ds(g * L, L)] & 7)
            pos_n = iota * 8 + (inn[pl.ds(g * L, L)] & 7)
            acc = jnp.zeros((L,), jnp.float32)
            for d in range(D):
                dtv = jnp.full((L,), d // 8, jnp.int32)
                sv = jnp.full((L,), d % 8, jnp.int32)
                uv = plsc.load_gather(gu, [bv, dtv, sv, pos_u])
                pv = plsc.load_gather(gp, [bv, dtv, sv, pos_p])
                nv = plsc.load_gather(gn, [bv, dtv, sv, pos_n])
                acc = acc + uv * (pv - nv)
            dv[pl.ds(g * L, L)] = acc
            return carry

        lax.fori_loop(0, NG, body, 0)
        pltpu.sync_copy(dv, out.at[sl])

    return k(user_t, item_t, user_ids, pos_ids, neg_ids)


def _tc_loss_kernel(x_ref, o_ref):
    o_ref[0, 0] = -jnp.sum(jax.nn.log_sigmoid(x_ref[:, :]))


def _tc_loss(diff):
    x = diff.reshape(B // 128, 128)
    res = pl.pallas_call(
        _tc_loss_kernel,
        out_shape=jax.ShapeDtypeStruct((1, 1), jnp.float32),
        out_specs=pl.BlockSpec(memory_space=pltpu.SMEM),
    )(x)
    return res[0, 0]


def kernel(user_emb, item_emb, user_ids, pos_ids, neg_ids):
    n_users = user_emb.shape[0]
    n_items = item_emb.shape[0]
    user_t = user_emb.T.reshape(D // 8, 8, n_users)
    item_t = item_emb.T.reshape(D // 8, 8, n_items)
    diff = _sc_diff(user_t, item_t, user_ids, pos_ids, neg_ids)
    return _tc_loss(diff)


# speedup vs baseline: 1.0095x; 1.0095x over previous
import functools

import jax
import jax.numpy as jnp
from jax import lax
from jax.experimental import pallas as pl
from jax.experimental.pallas import tpu as pltpu
from jax.experimental.pallas import tpu_sc as plsc

NC = 2
NS = 16
L = 16
NW = NC * NS
B = 16384
D = 32
BPW = B // NW
NG = BPW // L


def _sc_diff(user_t, item_t, user_ids, pos_ids, neg_ids):
    mesh = plsc.VectorSubcoreMesh(core_axis_name="c", subcore_axis_name="s")

    @functools.partial(
        pl.kernel,
        mesh=mesh,
        compiler_params=pltpu.CompilerParams(
            needs_layout_passes=False, use_tc_tiling_on_sc=True
        ),
        out_type=jax.ShapeDtypeStruct((B,), jnp.float32),
        scratch_types=[
            pltpu.VMEM((BPW,), jnp.int32),
            pltpu.VMEM((BPW,), jnp.int32),
            pltpu.VMEM((BPW,), jnp.int32),
            pltpu.VMEM((2, 4, 8, 128), jnp.float32),
            pltpu.VMEM((2, 4, 8, 128), jnp.float32),
            pltpu.VMEM((2, 4, 8, 128), jnp.float32),
            pltpu.VMEM((BPW,), jnp.float32),
            pltpu.SemaphoreType.DMA((2,)),
            pltpu.SemaphoreType.DMA((2,)),
            pltpu.SemaphoreType.DMA((2,)),
            pltpu.SemaphoreType.DMA,
        ],
    )
    def k(ut, it, uids, pids, nids, out, iu, ip, inn, gu, gp, gn, dv,
          usem, psem, nsem, isem):
        wid = lax.axis_index("s") * NC + lax.axis_index("c")
        base = wid * BPW
        sl = pl.ds(base, BPW)
        pltpu.async_copy(uids.at[sl], iu, isem)
        pltpu.async_copy(pids.at[sl], ip, isem)
        pltpu.async_copy(nids.at[sl], inn, isem)
        pltpu.make_async_copy(uids.at[sl], iu, isem).wait()
        pltpu.make_async_copy(pids.at[sl], ip, isem).wait()
        pltpu.make_async_copy(nids.at[sl], inn, isem).wait()

        def fire(g, b):
            cu = iu[pl.ds(g * L, L)]
            cp = ip[pl.ds(g * L, L)]
            cn = inn[pl.ds(g * L, L)]
            for i in range(L):
                dst = pl.ds(i * 8, 8)
                r0 = (cu[i] // 8) * 8
                pltpu.async_copy(
                    ut.at[:, :, pl.ds(r0, 8)],
                    gu.at[b, :, :, dst], usem.at[b])
                r0 = (cp[i] // 8) * 8
                pltpu.async_copy(
                    it.at[:, :, pl.ds(r0, 8)],
                    gp.at[b, :, :, dst], psem.at[b])
                r0 = (cn[i] // 8) * 8
                pltpu.async_copy(
                    it.at[:, :, pl.ds(r0, 8)],
                    gn.at[b, :, :, dst], nsem.at[b])

        fire(0, 0)
        iota = lax.iota(jnp.int32, L)

        def body(g, carry):
            b = lax.rem(g, 2)

            @pl.when(g < NG - 1)
            def _():
                fire(g + 1, 1 - b)

            dummy = ut.at[:, :, pl.ds(0, 128)]
            pltpu.make_async_copy(dummy, gu.at[b], usem.at[b]).wait()
            pltpu.make_async_copy(dummy, gp.at[b], psem.at[b]).wait()
            pltpu.make_async_copy(dummy, gn.at[b], nsem.at[b]).wait()

            bv = jnp.full((L,), b, jnp.int32)
            pos_u = iota * 8 + (iu[pl.ds(g * L, L)] & 7)
            pos_p = iota * 8 + (ip[pl.ds(g * L, L)] & 7)
            pos_n = iota * 8 + (inn[pl.ds(g * L, L)] & 7)
            acc = jnp.zeros((L,), jnp.float32)
            for d in range(D):
                dtv = jnp.full((L,), d // 8, jnp.int32)
                sv = jnp.full((L,), d % 8, jnp.int32)
                uv = plsc.load_gather(gu, [bv, dtv, sv, pos_u])
                pv = plsc.load_gather(gp, [bv, dtv, sv, pos_p])
                nv = plsc.load_gather(gn, [bv, dtv, sv, pos_n])
                acc = acc + uv * (pv - nv)
            dv[pl.ds(g * L, L)] = acc
            return carry

        lax.fori_loop(0, NG, body, 0)
        pltpu.sync_copy(dv, out.at[sl])

    return k(user_t, item_t, user_ids, pos_ids, neg_ids)


def _tc_loss_kernel(x_ref, o_ref):
    o_ref[0, 0] = -jnp.sum(jax.nn.log_sigmoid(x_ref[:, :]))


def _tc_loss(diff):
    x = diff.reshape(B // 128, 128)
    res = pl.pallas_call(
        _tc_loss_kernel,
        out_shape=jax.ShapeDtypeStruct((1, 1), jnp.float32),
        out_specs=pl.BlockSpec(memory_space=pltpu.SMEM),
    )(x)
    return res[0, 0]


def kernel(user_emb, item_emb, user_ids, pos_ids, neg_ids):
    n_users = user_emb.shape[0]
    n_items = item_emb.shape[0]
    user_t = user_emb.T.reshape(D // 8, 8, n_users)
    item_t = item_emb.T.reshape(D // 8, 8, n_items)
    diff = _sc_diff(user_t, item_t, user_ids, pos_ids, neg_ids)
    return _tc_loss(diff)


# speedup vs baseline: 1.1396x; 1.1288x over previous
import functools

import jax
import jax.numpy as jnp
from jax import lax
from jax.experimental import pallas as pl
from jax.experimental.pallas import tpu as pltpu
from jax.experimental.pallas import tpu_sc as plsc

NC = 2
NS = 16
L = 16
NW = NC * NS
B = 16384
D = 32
BPW = B // NW
NG = BPW // L
NJ = BPW // 128


def _sc_user(user_lin, user_ids):
    mesh = plsc.VectorSubcoreMesh(core_axis_name="c", subcore_axis_name="s")

    @functools.partial(
        pl.kernel,
        mesh=mesh,
        compiler_params=pltpu.CompilerParams(
            needs_layout_passes=False, use_tc_tiling_on_sc=False
        ),
        out_type=jax.ShapeDtypeStruct((D, B), jnp.float32),
        scratch_types=[
            pltpu.VMEM((NJ, 128), jnp.int32),
            pltpu.VMEM((2 * BPW,), jnp.float32),
            pltpu.SemaphoreType.DMA((2,)),
            pltpu.SemaphoreType.DMA,
        ],
    )
    def k(ut, uids, out, iu, ub, usem, isem):
        wid = lax.axis_index("s") * NC + lax.axis_index("c")
        base = wid * BPW
        for j in range(NJ):
            pltpu.async_copy(uids.at[pl.ds(base + j * 128, 128)], iu.at[j], isem)
        for j in range(NJ):
            pltpu.make_async_copy(
                uids.at[pl.ds(base + j * 128, 128)], iu.at[j], isem).wait()

        def fire(d, b):
            for j in range(NJ):
                pltpu.async_copy(
                    ut.at[d].at[iu.at[j]],
                    ub.at[pl.ds(b * BPW + j * 128, 128)], usem.at[b])

        def drain(d, b):
            for j in range(NJ):
                pltpu.make_async_copy(
                    ut.at[d].at[iu.at[j]],
                    ub.at[pl.ds(b * BPW + j * 128, 128)], usem.at[b]).wait()

        fire(0, 0)

        def body(d, carry):
            b = lax.rem(d, 2)

            @pl.when(d < D - 1)
            def _():
                fire(d + 1, 1 - b)

            drain(d, b)
            pltpu.sync_copy(
                ub.at[pl.ds(b * BPW, BPW)], out.at[d, pl.ds(base, BPW)])
            return carry

        lax.fori_loop(0, D, body, 0)

    return k(user_lin, user_ids)


def _sc_item(item_t, u_vals, pos_ids, neg_ids):
    mesh = plsc.VectorSubcoreMesh(core_axis_name="c", subcore_axis_name="s")

    @functools.partial(
        pl.kernel,
        mesh=mesh,
        compiler_params=pltpu.CompilerParams(
            needs_layout_passes=False, use_tc_tiling_on_sc=True
        ),
        out_type=jax.ShapeDtypeStruct((B,), jnp.float32),
        scratch_types=[
            pltpu.VMEM((BPW,), jnp.int32),
            pltpu.VMEM((BPW,), jnp.int32),
            pltpu.VMEM((D, BPW), jnp.float32),
            pltpu.VMEM((2, 4, 8, 128), jnp.float32),
            pltpu.VMEM((2, 4, 8, 128), jnp.float32),
            pltpu.VMEM((BPW,), jnp.float32),
            pltpu.SemaphoreType.DMA((2,)),
            pltpu.SemaphoreType.DMA((2,)),
            pltpu.SemaphoreType.DMA,
        ],
    )
    def k(it, uv, pids, nids, out, ip, inn, ub2, gp, gn, dv, psem, nsem, isem):
        wid = lax.axis_index("s") * NC + lax.axis_index("c")
        base = wid * BPW
        sl = pl.ds(base, BPW)
        pltpu.async_copy(pids.at[sl], ip, isem)
        pltpu.async_copy(nids.at[sl], inn, isem)
        pltpu.async_copy(uv.at[:, sl], ub2, isem)
        pltpu.make_async_copy(pids.at[sl], ip, isem).wait()
        pltpu.make_async_copy(nids.at[sl], inn, isem).wait()
        pltpu.make_async_copy(uv.at[:, sl], ub2, isem).wait()

        def fire(g, b):
            cp = ip[pl.ds(g * L, L)]
            cn = inn[pl.ds(g * L, L)]
            for i in range(L):
                dst = pl.ds(i * 8, 8)
                r0 = (cp[i] // 8) * 8
                pltpu.async_copy(
                    it.at[:, :, pl.ds(r0, 8)],
                    gp.at[b, :, :, dst], psem.at[b])
                r0 = (cn[i] // 8) * 8
                pltpu.async_copy(
                    it.at[:, :, pl.ds(r0, 8)],
                    gn.at[b, :, :, dst], nsem.at[b])

        fire(0, 0)
        iota = lax.iota(jnp.int32, L)

        def body(g, carry):
            b = lax.rem(g, 2)

            @pl.when(g < NG - 1)
            def _():
                fire(g + 1, 1 - b)

            dummy = it.at[:, :, pl.ds(0, 128)]
            pltpu.make_async_copy(dummy, gp.at[b], psem.at[b]).wait()
            pltpu.make_async_copy(dummy, gn.at[b], nsem.at[b]).wait()

            bv = jnp.full((L,), b, jnp.int32)
            pos_p = iota * 8 + (ip[pl.ds(g * L, L)] & 7)
            pos_n = iota * 8 + (inn[pl.ds(g * L, L)] & 7)
            acc = jnp.zeros((L,), jnp.float32)
            for d in range(D):
                dtv = jnp.full((L,), d // 8, jnp.int32)
                sv = jnp.full((L,), d % 8, jnp.int32)
                pv = plsc.load_gather(gp, [bv, dtv, sv, pos_p])
                nv = plsc.load_gather(gn, [bv, dtv, sv, pos_n])
                acc = acc + ub2[d, pl.ds(g * L, L)] * (pv - nv)
            dv[pl.ds(g * L, L)] = acc
            return carry

        lax.fori_loop(0, NG, body, 0)
        pltpu.sync_copy(dv, out.at[sl])

    return k(item_t, u_vals, pos_ids, neg_ids)


def _tc_loss_kernel(x_ref, o_ref):
    o_ref[0, 0] = -jnp.sum(jax.nn.log_sigmoid(x_ref[:, :]))


def _tc_loss(diff):
    x = diff.reshape(B // 128, 128)
    res = pl.pallas_call(
        _tc_loss_kernel,
        out_shape=jax.ShapeDtypeStruct((1, 1), jnp.float32),
        out_specs=pl.BlockSpec(memory_space=pltpu.SMEM),
    )(x)
    return res[0, 0]


def kernel(user_emb, item_emb, user_ids, pos_ids, neg_ids):
    n_items = item_emb.shape[0]
    item_t = item_emb.T.reshape(D // 8, 8, n_items)
    u_vals = _sc_user(user_emb.T, user_ids)
    diff = _sc_item(item_t, u_vals, pos_ids, neg_ids)
    return _tc_loss(diff)
